# baseline (speedup 1.0000x reference)
import functools

import jax
import jax.numpy as jnp
from jax import lax
from jax.experimental import pallas as pl
from jax.experimental.pallas import tpu as pltpu
from jax.experimental.pallas import tpu_sc as plsc

WORLD_SIZE = 4
HALO = 8192
B = WORLD_SIZE * HALO
D = 256

_info = plsc.get_sparse_core_info()
NC = _info.num_cores
NS = _info.num_subcores
NW = NC * NS
B_PER_W = B // NW
CHUNK = 64
NCHUNK = B_PER_W // CHUNK
NB = 2
NB2 = 2


def _sc_gather(table, idx2d):
    mesh = plsc.VectorSubcoreMesh(core_axis_name="c", subcore_axis_name="s")

    @functools.partial(
        pl.kernel,
        mesh=mesh,
        out_type=jax.ShapeDtypeStruct((B, D), jnp.float32),
        scratch_types=(
            [pltpu.VMEM((NCHUNK, CHUNK), jnp.int32)]
            + [pltpu.VMEM((CHUNK, D), jnp.float32) for _ in range(NB)]
            + [pltpu.VMEM_SHARED((NS, NB2, CHUNK, D), jnp.float32)]
            + [pltpu.SemaphoreType.DMA for _ in range(NB + 2 * NB2)]
        ),
    )
    def k(table_hbm, idx_hbm, out_hbm, idx_v, *rest):
        bufs = rest[:NB]
        spbuf = rest[NB]
        gsem = rest[NB + 1:NB + 1 + NB]
        msem = rest[NB + 1 + NB:NB + 1 + NB + NB2]
        osem = rest[NB + 1 + NB + NB2:]
        sid = lax.axis_index("s")
        wid = sid * NC + lax.axis_index("c")
        base = wid * B_PER_W

        pltpu.sync_copy(idx_hbm.at[pl.ds(wid * NCHUNK, NCHUNK)], idx_v)

        def start_gather(j):
            return pltpu.async_copy(
                table_hbm.at[idx_v.at[j]], bufs[j % NB], gsem[j % NB])

        def start_move(j):
            return pltpu.async_copy(
                bufs[j % NB], spbuf.at[sid, j % NB2], msem[j % NB2])

        def start_out(j):
            return pltpu.async_copy(
                spbuf.at[sid, j % NB2],
                out_hbm.at[pl.ds(base + j * CHUNK, CHUNK)], osem[j % NB2])

        gathers = [None] * NCHUNK
        moves = [None] * NCHUNK
        outs = [None] * NCHUNK
        for j in range(min(NB, NCHUNK)):
            gathers[j] = start_gather(j)
        for j in range(NCHUNK):
            gathers[j].wait()
            moves[j] = start_move(j)
            moves[j].wait()
            if j + NB < NCHUNK:
                gathers[j + NB] = start_gather(j + NB)
            outs[j] = start_out(j)
            if j + NB2 < NCHUNK:
                outs[j].wait()
        for j in range(max(0, NCHUNK - NB2), NCHUNK):
            outs[j].wait()

    return k(table, idx2d)


def kernel(local, lidx):
    return _sc_gather(local, lidx.reshape(B // CHUNK, CHUNK))


# speedup vs baseline: 1.0706x; 1.0706x over previous
import functools

import jax
import jax.numpy as jnp
from jax import lax
from jax.experimental import pallas as pl
from jax.experimental.pallas import tpu as pltpu
from jax.experimental.pallas import tpu_sc as plsc

WORLD_SIZE = 4
HALO = 8192
B = WORLD_SIZE * HALO
D = 256

_info = plsc.get_sparse_core_info()
NC = _info.num_cores
NS = _info.num_subcores
NW = NC * NS
B_PER_W = B // NW
CHUNK = 64
NCHUNK = B_PER_W // CHUNK
NBUF = 7


def _sc_gather(table, idx2d):
    mesh = plsc.VectorSubcoreMesh(core_axis_name="c", subcore_axis_name="s")

    @functools.partial(
        pl.kernel,
        mesh=mesh,
        out_type=jax.ShapeDtypeStruct((B, D), jnp.float32),
        scratch_types=(
            [pltpu.VMEM((NCHUNK, CHUNK), jnp.int32)]
            + [pltpu.VMEM((CHUNK, D), jnp.float32) for _ in range(NBUF)]
            + [pltpu.SemaphoreType.DMA for _ in range(2 * NBUF)]
        ),
    )
    def k(table_hbm, idx_hbm, out_hbm, idx_v, *bufs_and_sems):
        bufs = bufs_and_sems[:NBUF]
        gsem = bufs_and_sems[NBUF:2 * NBUF]
        osem = bufs_and_sems[2 * NBUF:]
        wid = lax.axis_index("s") * NC + lax.axis_index("c")
        base = wid * B_PER_W

        head = 8
        pltpu.sync_copy(idx_hbm.at[pl.ds(wid * NCHUNK, head)],
                        idx_v.at[pl.ds(0, head)])

        def start_gather(j):
            return pltpu.async_copy(
                table_hbm.at[idx_v.at[j]], bufs[j % NBUF], gsem[j % NBUF])

        def start_out(j):
            return pltpu.async_copy(
                bufs[j % NBUF], out_hbm.at[pl.ds(base + j * CHUNK, CHUNK)],
                osem[j % NBUF])

        gathers = [None] * NCHUNK
        for j in range(min(NBUF, NCHUNK)):
            gathers[j] = start_gather(j)
        if head < NCHUNK:
            pltpu.sync_copy(idx_hbm.at[pl.ds(wid * NCHUNK + head, NCHUNK - head)],
                            idx_v.at[pl.ds(head, NCHUNK - head)])
        outs = [None] * NCHUNK
        for j in range(NCHUNK):
            gathers[j].wait()
            outs[j] = start_out(j)
            if j + NBUF < NCHUNK:
                outs[j].wait()
                gathers[j + NBUF] = start_gather(j + NBUF)
        for j in range(max(0, NCHUNK - NBUF), NCHUNK):
            outs[j].wait()

    return k(table, idx2d)


def kernel(local, lidx):
    return _sc_gather(local, lidx.reshape(B // CHUNK, CHUNK))
